# baseline (speedup 1.0000x reference)
import functools

import jax
import jax.numpy as jnp
from jax import lax
from jax.experimental import pallas as pl
from jax.experimental.pallas import tpu as pltpu
from jax.experimental.pallas import tpu_sc as plsc

LANES = 16
TPW = 128
BLK = 256


def _route_kernel(s_hbm, x_hbm, xg_hbm, be_hbm, bvc_hbm, cnts_hbm,
                  sv_ref, xrows_ref, slots_ref, cnt_ref, callf_ref,
                  tbl_ref, sem, *, n_exp, n_sub, a_len, d_model):
    c = lax.axis_index("c")
    s = lax.axis_index("s")
    wid = c * n_sub + s
    base = wid * TPW
    nblk_mol = (a_len // BLK) + n_exp
    slot_base = c * (nblk_mol * BLK)
    lane = lax.iota(jnp.int32, LANES)
    n_chunks = TPW // LANES

    pltpu.sync_copy(s_hbm.at[pl.ds(base, TPW)], sv_ref)
    pltpu.sync_copy(x_hbm.at[pl.ds(base, TPW)], xrows_ref)

    lc = jnp.zeros((LANES,), jnp.int32)
    for ch in range(n_chunks):
        sv = sv_ref[pl.ds(ch * LANES, LANES)]
        for e in range(n_exp):
            pc = jnp.sum((sv == e).astype(jnp.int32))
            lc = jnp.where(lane == e, lc + pc, lc)
    cnt_ref[...] = lc

    pltpu.sync_copy(cnt_ref, cnts_hbm.at[pl.ds(wid * LANES, LANES)])
    plsc.subcore_barrier()
    pltpu.sync_copy(cnts_hbm.at[pl.ds(c * (n_sub * LANES), n_sub * LANES)],
                    callf_ref)

    tb = []
    ct = []
    bstart = []
    blk_cursor = jnp.int32(0)
    for e in range(n_exp):
        col = plsc.load_gather(callf_ref, [lane * LANES + e])
        ct_e = jnp.sum(col)
        prefix_excl = plsc.cumsum(col) - col
        mine = jnp.sum(jnp.where(lane == s, prefix_excl, 0))
        tb.append(slot_base + blk_cursor * BLK + mine)
        ct.append(ct_e)
        bstart.append(blk_cursor)
        blk_cursor = blk_cursor + (ct_e + (BLK - 1)) // BLK

    running = [jnp.full((LANES,), tb[e], jnp.int32) for e in range(n_exp)]
    for ch in range(n_chunks):
        sv = sv_ref[pl.ds(ch * LANES, LANES)]
        slot = jnp.zeros((LANES,), jnp.int32)
        for e in range(n_exp):
            m = sv == e
            csum = plsc.cumsum(m.astype(jnp.int32))
            slot = jnp.where(m, running[e] + csum - 1, slot)
            running[e] = running[e] + jnp.sum(m.astype(jnp.int32))
        slots_ref[pl.ds(ch * LANES, LANES)] = slot

    pltpu.async_copy(xrows_ref, xg_hbm.at[slots_ref], sem).wait()

    @pl.when(s == 0)
    def _():
        for half in range(2):
            blkvec = lane + half * LANES
            evec = jnp.zeros((LANES,), jnp.int32)
            vcvec = jnp.zeros((LANES,), jnp.int32)
            for e in range(n_exp):
                pe = (ct[e] + (BLK - 1)) // BLK
                m = (blkvec >= bstart[e]) & (blkvec < bstart[e] + pe)
                evec = jnp.where(m, e, evec)
                vc = jnp.clip(ct[e] - (blkvec - bstart[e]) * BLK, 0, BLK)
                vcvec = jnp.where(m, vc, vcvec)
            tbl_ref[pl.ds(half * LANES, LANES)] = evec
            tbl_ref[pl.ds(32 + half * LANES, LANES)] = vcvec
        pltpu.sync_copy(tbl_ref.at[pl.ds(0, 32)], be_hbm.at[pl.ds(c * 32, 32)])
        pltpu.sync_copy(tbl_ref.at[pl.ds(32, 32)], bvc_hbm.at[pl.ds(c * 32, 32)])


def _mlp_kernel(be_ref, bvc_ref, xg_ref, w1_ref, b1_ref, w2_ref, b2_ref,
                w3_ref, b3_ref, w4_ref, b4_ref, out_ref, acc_ref,
                *, nblk_mol, nblk):
    t = pl.program_id(0)

    @pl.when(t == 0)
    def _():
        for m in range(out_ref.shape[0]):
            acc_ref[m] = 0.0

    vc = bvc_ref[t]

    @pl.when(vc > 0)
    def _():
        x = xg_ref[...]
        h = jnp.maximum(
            lax.dot_general(x, w1_ref[0], (((1,), (0,)), ((), ())),
                            preferred_element_type=jnp.float32)
            + b1_ref[0], 0.0)
        h = jnp.maximum(
            lax.dot_general(h, w2_ref[0], (((1,), (0,)), ((), ())),
                            preferred_element_type=jnp.float32)
            + b2_ref[0], 0.0)
        h = jnp.maximum(
            lax.dot_general(h, w3_ref[0], (((1,), (0,)), ((), ())),
                            preferred_element_type=jnp.float32)
            + b3_ref[0], 0.0)
        rows = lax.broadcasted_iota(jnp.int32, h.shape, 0)
        hm = jnp.where(rows < vc, h, 0.0)
        ones = jnp.ones((1, hm.shape[0]), jnp.float32)
        csum = lax.dot_general(ones, hm, (((1,), (0,)), ((), ())),
                               preferred_element_type=jnp.float32)
        w4r = w4_ref[0][:, 0][None, :]
        total = jnp.sum(csum * w4r) \
            + vc.astype(jnp.float32) * b4_ref[0][0, 0]
        mol = t // nblk_mol
        acc_ref[mol] += total

    @pl.when(t == nblk - 1)
    def _():
        orow = lax.broadcasted_iota(jnp.int32, out_ref.shape, 0)
        res = jnp.zeros(out_ref.shape, jnp.float32)
        for m in range(out_ref.shape[0]):
            res = jnp.where(orow == m, acc_ref[m], res)
        out_ref[...] = res


def kernel(species, input, W1, b1, W2, b2, W3, b3, W4, b4):
    B, A = species.shape
    D = input.shape[-1]
    E = W1.shape[0]
    N = B * A
    nblk_mol = (A // BLK) + E
    nblk = B * nblk_mol
    nslots = nblk * BLK

    s_flat = species.reshape(-1).astype(jnp.int32)
    x_flat = input.reshape(N, D)

    mesh = plsc.VectorSubcoreMesh(core_axis_name="c", subcore_axis_name="s")
    route = functools.partial(
        pl.kernel,
        out_type=[
            jax.ShapeDtypeStruct((nslots, D), jnp.float32),
            jax.ShapeDtypeStruct((B * 32,), jnp.int32),
            jax.ShapeDtypeStruct((B * 32,), jnp.int32),
            jax.ShapeDtypeStruct((B * LANES * LANES,), jnp.int32),
        ],
        mesh=mesh,
        compiler_params=pltpu.CompilerParams(needs_layout_passes=False),
        scratch_types=[
            pltpu.VMEM((TPW,), jnp.int32),
            pltpu.VMEM((TPW, D), jnp.float32),
            pltpu.VMEM((TPW,), jnp.int32),
            pltpu.VMEM((LANES,), jnp.int32),
            pltpu.VMEM((LANES * LANES,), jnp.int32),
            pltpu.VMEM((64,), jnp.int32),
            pltpu.SemaphoreType.DMA,
        ],
    )(functools.partial(_route_kernel, n_exp=E, n_sub=LANES, a_len=A,
                        d_model=D))
    xg, be, bvc, _ = route(s_flat, x_flat)

    be_flat = be.reshape(B, 32)[:, :nblk_mol].reshape(-1)
    bvc_flat = bvc.reshape(B, 32)[:, :nblk_mol].reshape(-1)

    def wspec(shape):
        return pl.BlockSpec((1,) + shape[1:],
                            lambda t, be_, bvc_: (be_[t],) + (0,) * (len(shape) - 1))

    def xg_map(t, be_, bvc_):
        return (jnp.where(bvc_[t] > 0, t, (t // nblk_mol) * nblk_mol), 0)

    b1r, b2r, b3r, b4r = (b[:, None, :] for b in (b1, b2, b3, b4))
    grid_spec = pltpu.PrefetchScalarGridSpec(
        num_scalar_prefetch=2,
        grid=(nblk,),
        in_specs=[
            pl.BlockSpec((BLK, D), xg_map),
            wspec(W1.shape), wspec(b1r.shape),
            wspec(W2.shape), wspec(b2r.shape),
            wspec(W3.shape), wspec(b3r.shape),
            wspec(W4.shape), wspec(b4r.shape),
        ],
        out_specs=pl.BlockSpec((B, 1), lambda t, be_, bvc_: (0, 0)),
        scratch_shapes=[pltpu.SMEM((B,), jnp.float32)],
    )
    out = pl.pallas_call(
        functools.partial(_mlp_kernel, nblk_mol=nblk_mol, nblk=nblk),
        grid_spec=grid_spec,
        out_shape=jax.ShapeDtypeStruct((B, 1), jnp.float32),
    )(be_flat, bvc_flat, xg, W1, b1r, W2, b2r, W3, b3r, W4, b4r)
    return out


# speedup vs baseline: 1.0713x; 1.0713x over previous
import functools

import jax
import jax.numpy as jnp
from jax import lax
from jax.experimental import pallas as pl
from jax.experimental.pallas import tpu as pltpu
from jax.experimental.pallas import tpu_sc as plsc

LANES = 16
TPW = 128
BLK = 256


def _route_kernel(s_hbm, x_hbm, xg_hbm, be_hbm, bvc_hbm, cnts_hbm,
                  sv_ref, xrows_ref, slots_ref, cnt_ref, callf_ref,
                  tbl_ref, sem, *, n_exp, n_sub, a_len):
    c = lax.axis_index("c")
    s = lax.axis_index("s")
    wid = c * n_sub + s
    nblk_mol = (a_len // BLK) + n_exp
    slot_base = c * (nblk_mol * BLK)
    lane = lax.iota(jnp.int32, LANES)
    n_chunks = TPW // LANES

    pltpu.sync_copy(s_hbm.at[c, pl.ds(s * TPW, TPW)], sv_ref)
    pltpu.sync_copy(x_hbm.at[c, pl.ds(s * TPW, TPW)], xrows_ref)

    lc = jnp.zeros((LANES,), jnp.int32)
    for ch in range(n_chunks):
        sv = sv_ref[pl.ds(ch * LANES, LANES)]
        for e in range(n_exp):
            pc = jnp.sum((sv == e).astype(jnp.int32))
            lc = jnp.where(lane == e, lc + pc, lc)
    cnt_ref[...] = lc

    pltpu.sync_copy(cnt_ref, cnts_hbm.at[pl.ds(wid * LANES, LANES)])
    plsc.subcore_barrier()
    pltpu.sync_copy(cnts_hbm.at[pl.ds(c * (n_sub * LANES), n_sub * LANES)],
                    callf_ref)

    tb = []
    ct = []
    bstart = []
    blk_cursor = jnp.int32(0)
    for e in range(n_exp):
        col = plsc.load_gather(callf_ref, [lane * LANES + e])
        ct_e = jnp.sum(col)
        prefix_excl = plsc.cumsum(col) - col
        mine = jnp.sum(jnp.where(lane == s, prefix_excl, 0))
        tb.append(slot_base + blk_cursor * BLK + mine)
        ct.append(ct_e)
        bstart.append(blk_cursor)
        blk_cursor = blk_cursor + (ct_e + (BLK - 1)) // BLK

    running = [jnp.full((LANES,), tb[e], jnp.int32) for e in range(n_exp)]
    for ch in range(n_chunks):
        sv = sv_ref[pl.ds(ch * LANES, LANES)]
        slot = jnp.zeros((LANES,), jnp.int32)
        for e in range(n_exp):
            m = sv == e
            csum = plsc.cumsum(m.astype(jnp.int32))
            slot = jnp.where(m, running[e] + csum - 1, slot)
            running[e] = running[e] + jnp.sum(m.astype(jnp.int32))
        slots_ref[pl.ds(ch * LANES, LANES)] = slot

    pltpu.async_copy(xrows_ref, xg_hbm.at[slots_ref], sem).wait()

    @pl.when(s == 0)
    def _():
        blkvec = lane
        evec = jnp.zeros((LANES,), jnp.int32)
        vcvec = jnp.zeros((LANES,), jnp.int32)
        for e in range(n_exp):
            pe = (ct[e] + (BLK - 1)) // BLK
            m = (blkvec >= bstart[e]) & (blkvec < bstart[e] + pe)
            evec = jnp.where(m, e, evec)
            vc = jnp.clip(ct[e] - (blkvec - bstart[e]) * BLK, 0, BLK)
            vcvec = jnp.where(m, vc, vcvec)
        tbl_ref[pl.ds(0, LANES)] = evec
        tbl_ref[pl.ds(LANES, LANES)] = vcvec
        pltpu.sync_copy(tbl_ref.at[pl.ds(0, LANES)],
                        be_hbm.at[pl.ds(c * LANES, LANES)])
        pltpu.sync_copy(tbl_ref.at[pl.ds(LANES, LANES)],
                        bvc_hbm.at[pl.ds(c * LANES, LANES)])


def _mlp_kernel(be_ref, bvc_ref, xg_ref, w1_ref, b1_ref, w2_ref, b2_ref,
                w3_ref, b3_ref, w4_ref, b4_ref, out_ref, acc_ref,
                *, nblk_mol, nblk):
    t = pl.program_id(0)

    @pl.when(t == 0)
    def _():
        for m in range(out_ref.shape[0]):
            acc_ref[m] = 0.0

    vc = bvc_ref[t]

    @pl.when(vc > 0)
    def _():
        e = be_ref[t]
        x = xg_ref[...]
        h = jnp.maximum(
            lax.dot_general(x, w1_ref[0], (((1,), (0,)), ((), ())),
                            preferred_element_type=jnp.float32)
            + b1_ref[e][None, :], 0.0)
        h = jnp.maximum(
            lax.dot_general(h, w2_ref[e], (((1,), (0,)), ((), ())),
                            preferred_element_type=jnp.float32)
            + b2_ref[e][None, :], 0.0)
        h = jnp.maximum(
            lax.dot_general(h, w3_ref[e], (((1,), (0,)), ((), ())),
                            preferred_element_type=jnp.float32)
            + b3_ref[e][None, :], 0.0)
        rows = lax.broadcasted_iota(jnp.int32, h.shape, 0)
        hm = jnp.where(rows < vc, h, 0.0)
        ones = jnp.ones((1, BLK), jnp.float32)
        csum = lax.dot_general(ones, hm, (((1,), (0,)), ((), ())),
                               preferred_element_type=jnp.float32)
        w4r = w4_ref[e][:, 0][None, :]
        total = jnp.sum(csum * w4r) \
            + vc.astype(jnp.float32) * b4_ref[e, 0]
        mol = t // nblk_mol
        acc_ref[mol] += total

    @pl.when(t == nblk - 1)
    def _():
        orow = lax.broadcasted_iota(jnp.int32, out_ref.shape, 0)
        res = jnp.zeros(out_ref.shape, jnp.float32)
        for m in range(out_ref.shape[0]):
            res = jnp.where(orow == m, acc_ref[m], res)
        out_ref[...] = res


def kernel(species, input, W1, b1, W2, b2, W3, b3, W4, b4):
    B, A = species.shape
    D = input.shape[-1]
    E = W1.shape[0]
    nblk_mol = (A // BLK) + E
    nblk = B * nblk_mol
    nslots = nblk * BLK

    sp = species.astype(jnp.int32)

    mesh = plsc.VectorSubcoreMesh(core_axis_name="c", subcore_axis_name="s")
    route = functools.partial(
        pl.kernel,
        out_type=[
            jax.ShapeDtypeStruct((nslots, D), jnp.float32),
            jax.ShapeDtypeStruct((B * LANES,), jnp.int32),
            jax.ShapeDtypeStruct((B * LANES,), jnp.int32),
            jax.ShapeDtypeStruct((B * LANES * LANES,), jnp.int32),
        ],
        mesh=mesh,
        compiler_params=pltpu.CompilerParams(needs_layout_passes=False),
        scratch_types=[
            pltpu.VMEM((TPW,), jnp.int32),
            pltpu.VMEM((TPW, D), jnp.float32),
            pltpu.VMEM((TPW,), jnp.int32),
            pltpu.VMEM((LANES,), jnp.int32),
            pltpu.VMEM((LANES * LANES,), jnp.int32),
            pltpu.VMEM((2 * LANES,), jnp.int32),
            pltpu.SemaphoreType.DMA,
        ],
    )(functools.partial(_route_kernel, n_exp=E, n_sub=LANES, a_len=A))
    xg, be, bvc, _ = route(sp, input)

    def wspec(shape):
        return pl.BlockSpec((1,) + shape[1:],
                            lambda t, be_, bvc_: (be_[t],) + (0,) * (len(shape) - 1))

    def xg_map(t, be_, bvc_):
        return (jnp.where(bvc_[t] > 0, t, (t // nblk_mol) * nblk_mol), 0)

    whole = lambda shape: pl.BlockSpec(shape, lambda t, be_, bvc_: (0,) * len(shape))
    grid_spec = pltpu.PrefetchScalarGridSpec(
        num_scalar_prefetch=2,
        grid=(nblk,),
        in_specs=[
            pl.BlockSpec((BLK, D), xg_map),
            wspec(W1.shape), whole(b1.shape),
            whole(W2.shape), whole(b2.shape),
            whole(W3.shape), whole(b3.shape),
            whole(W4.shape), whole(b4.shape),
        ],
        out_specs=pl.BlockSpec((B, 1), lambda t, be_, bvc_: (0, 0)),
        scratch_shapes=[pltpu.SMEM((B,), jnp.float32)],
    )
    out = pl.pallas_call(
        functools.partial(_mlp_kernel, nblk_mol=nblk_mol, nblk=nblk),
        grid_spec=grid_spec,
        out_shape=jax.ShapeDtypeStruct((B, 1), jnp.float32),
    )(be, bvc, xg, W1, b1, W2, b2, W3, b3, W4, b4)
    return out
